# baseline (speedup 1.0000x reference)
import functools

import jax
import jax.numpy as jnp
from jax import lax
from jax.experimental import pallas as pl
from jax.experimental.pallas import tpu as pltpu
from jax.experimental.pallas import tpu_sc as plsc

N_NODES = 10000
LANES = 16
NUM_CORES = 2
NUM_SUBCORES = 16
NUM_WORKERS = NUM_CORES * NUM_SUBCORES
NPW = 320


@functools.partial(jax.jit, static_argnums=(2, 3))
def _mean_agg(neighs, table, k_nb, d_feat):
    inv_k = jnp.float32(1.0 / k_nb)
    epw = k_nb * NPW

    mesh = plsc.VectorSubcoreMesh(
        core_axis_name="c", subcore_axis_name="s", num_cores=NUM_CORES,
        num_subcores=NUM_SUBCORES)

    @functools.partial(
        pl.kernel,
        out_type=jax.ShapeDtypeStruct((N_NODES, d_feat), jnp.float32),
        mesh=mesh,
        compiler_params=pltpu.CompilerParams(needs_layout_passes=False),
        scratch_types=[
            pltpu.VMEM((epw,), jnp.int32),
            pltpu.VMEM((epw,), jnp.int32),
            pltpu.VMEM((NPW, d_feat), jnp.float32),
            pltpu.SemaphoreType.DMA,
        ],
    )
    def k(neighs_hbm, table_hbm, out_hbm, idx_nm, idxt_v, acc_v, sem):
        wid = lax.axis_index("s") * NUM_CORES + lax.axis_index("c")
        start = jnp.minimum(wid * NPW, N_NODES - NPW)

        pltpu.async_copy(neighs_hbm.at[pl.ds(start * k_nb, epw)], idx_nm, sem)

        zeros = jnp.zeros((LANES,), jnp.float32)

        def zero_body(c, _):
            for d in range(d_feat // LANES):
                acc_v[c, pl.ds(d * LANES, LANES)] = zeros
            return 0
        lax.fori_loop(0, NPW, zero_body, 0, unroll=8)

        pltpu.make_async_copy(neighs_hbm.at[pl.ds(0, epw)], idx_nm, sem).wait()

        lanes_npw = lax.broadcasted_iota(jnp.int32, (LANES,), 0) * NPW

        def t_body(e0, _):
            vec = idx_nm[pl.ds(e0 * LANES, LANES)]
            base = (e0 % 2) * (LANES * NPW) + e0 // 2
            plsc.store_scatter(idxt_v, [lanes_npw + base], vec)
            return 0
        lax.fori_loop(0, epw // LANES, t_body, 0, unroll=8)

        def fire_body(j, _):
            pltpu.async_copy(table_hbm.at[idxt_v.at[pl.ds(j * NPW, NPW)]],
                             acc_v, sem, add=True)
            return 0
        lax.fori_loop(0, k_nb, fire_body, 0, unroll=8)

        def drain_body(j, _):
            pltpu.make_async_copy(table_hbm.at[idxt_v.at[pl.ds(0, NPW)]],
                                  acc_v, sem).wait()
            return 0
        lax.fori_loop(0, k_nb, drain_body, 0, unroll=8)

        for d in range(d_feat // LANES):
            sl = pl.ds(d * LANES, LANES)

            def scale_body(c, _):
                acc_v[c, sl] = acc_v[c, sl] * inv_k
                return 0
            lax.fori_loop(0, NPW, scale_body, 0, unroll=8)
        pltpu.sync_copy(acc_v, out_hbm.at[pl.ds(start, NPW)])

    return k(neighs, table)


def kernel(neighs, node_count, table):
    del node_count
    k_nb = neighs.shape[0] // N_NODES
    return _mean_agg(neighs.astype(jnp.int32), table, k_nb, table.shape[1])


# speedup vs baseline: 1.0236x; 1.0236x over previous
import functools

import jax
import jax.numpy as jnp
from jax import lax
from jax.experimental import pallas as pl
from jax.experimental.pallas import tpu as pltpu
from jax.experimental.pallas import tpu_sc as plsc

N_NODES = 10000
LANES = 16
NUM_CORES = 2
NUM_SUBCORES = 16
NUM_WORKERS = NUM_CORES * NUM_SUBCORES
NPW = 320
NPH = NPW // 2


@functools.partial(jax.jit, static_argnums=(2, 3))
def _mean_agg(neighs, table, k_nb, d_feat):
    inv_k = jnp.float32(1.0 / k_nb)
    epw = k_nb * NPW
    eph = k_nb * NPH

    mesh = plsc.VectorSubcoreMesh(
        core_axis_name="c", subcore_axis_name="s", num_cores=NUM_CORES,
        num_subcores=NUM_SUBCORES)

    @functools.partial(
        pl.kernel,
        out_type=jax.ShapeDtypeStruct((N_NODES, d_feat), jnp.float32),
        mesh=mesh,
        compiler_params=pltpu.CompilerParams(needs_layout_passes=False),
        scratch_types=[
            pltpu.VMEM((epw,), jnp.int32),
            pltpu.VMEM((epw,), jnp.int32),
            pltpu.VMEM((NPH, d_feat), jnp.float32),
            pltpu.VMEM((NPH, d_feat), jnp.float32),
            pltpu.SemaphoreType.DMA,
            pltpu.SemaphoreType.DMA,
            pltpu.SemaphoreType.DMA,
            pltpu.SemaphoreType.DMA,
        ],
    )
    def k(neighs_hbm, table_hbm, out_hbm, idx_nm, idxt_v, acc_a, acc_b,
          sem_s, sem_a, sem_b, sem_o):
        wid = lax.axis_index("s") * NUM_CORES + lax.axis_index("c")
        start = jnp.minimum(wid * NPW, N_NODES - NPW)
        zeros = jnp.zeros((LANES,), jnp.float32)
        lanes_nph = lax.broadcasted_iota(jnp.int32, (LANES,), 0) * NPH
        dv = d_feat // LANES

        pltpu.sync_copy(neighs_hbm.at[pl.ds(start * k_nb, epw)], idx_nm)

        def prep_half(h, acc, sem):
            def t_body(e0, _):
                vec = idx_nm[pl.ds(e0 * LANES, LANES)]
                base = (h * eph + (e0 % 2) * (LANES * NPH)
                        + (e0 // 2 - h * NPH))
                plsc.store_scatter(idxt_v, [lanes_nph + base], vec)
                return 0
            lax.fori_loop(h * 2 * NPH, (h + 1) * 2 * NPH, t_body, 0,
                          unroll=8)

            def zero_body(c, _):
                for d in range(dv):
                    acc[c, pl.ds(d * LANES, LANES)] = zeros
                return 0
            lax.fori_loop(0, NPH, zero_body, 0, unroll=8)

            def fire_body(j, _):
                pltpu.async_copy(
                    table_hbm.at[idxt_v.at[pl.ds(h * eph + j * NPH, NPH)]],
                    acc, sem, add=True)
                return 0
            lax.fori_loop(0, k_nb, fire_body, 0)

        def finish_half(h, acc, sem):
            def drain_body(j, _):
                pltpu.make_async_copy(
                    table_hbm.at[idxt_v.at[pl.ds(0, NPH)]], acc, sem).wait()
                return 0
            lax.fori_loop(0, k_nb, drain_body, 0)

            for d in range(dv):
                sl = pl.ds(d * LANES, LANES)

                def scale_body(c, _):
                    acc[c, sl] = acc[c, sl] * inv_k
                    return 0
                lax.fori_loop(0, NPH, scale_body, 0, unroll=8)
            pltpu.async_copy(acc, out_hbm.at[pl.ds(start + h * NPH, NPH)],
                             sem_o)

        prep_half(0, acc_a, sem_a)
        prep_half(1, acc_b, sem_b)
        finish_half(0, acc_a, sem_a)
        finish_half(1, acc_b, sem_b)
        pltpu.make_async_copy(acc_a, out_hbm.at[pl.ds(0, NPH)], sem_o).wait()
        pltpu.make_async_copy(acc_b, out_hbm.at[pl.ds(0, NPH)], sem_o).wait()

    return k(neighs, table)


def kernel(neighs, node_count, table):
    del node_count
    k_nb = neighs.shape[0] // N_NODES
    return _mean_agg(neighs.astype(jnp.int32), table, k_nb, table.shape[1])
